# Initial kernel scaffold; baseline (speedup 1.0000x reference)
import jax
import jax.numpy as jnp
from jax.experimental import pallas as pl


def kernel(edge_feature, tables):
    raise NotImplementedError("write your pallas kernel here")


# speedup vs baseline: 22.0340x; 22.0340x over previous
import jax
import jax.numpy as jnp
from jax import lax
from jax.experimental import pallas as pl
from jax.experimental.pallas import tpu as pltpu
from jax.experimental.pallas import tpu_sc as plsc

EMB = 128
NF = 4
NE = 10
NCOMB = NE ** NF
N_PIX = 8 * 256 * 256
NC = 2
NS = 16
NW = NC * NS
ROWS_W = N_PIX // NW
CHUNK = 128
STEPS = ROWS_W // CHUNK
M = N_PIX // 128
MB = 512


def _table_body(tab_ref, out_ref):
    rid = lax.broadcasted_iota(jnp.int32, (NCOMB, 1), 0)
    acc = jnp.zeros((NCOMB, EMB), jnp.float32)
    div = NCOMB // NE
    for f in range(NF):
        sub = (rid // div) % NE
        oh = (sub == lax.broadcasted_iota(jnp.int32, (NCOMB, NE), 1)).astype(jnp.float32)
        t = tab_ref[f]
        hi = t.astype(jnp.bfloat16).astype(jnp.float32)
        acc = acc + jnp.dot(oh, hi, preferred_element_type=jnp.float32)
        acc = acc + jnp.dot(oh, t - hi, preferred_element_type=jnp.float32)
        div //= NE
    out_ref[...] = acc


_table_call = pl.pallas_call(
    _table_body,
    out_shape=jax.ShapeDtypeStruct((NCOMB, EMB), jnp.float32),
)


def _cidx_body(ef_ref, out_ref):
    x = ef_ref[...]
    out_ref[...] = ((x[0] * NE + x[1]) * NE + x[2]) * NE + x[3]


_cidx_call = pl.pallas_call(
    _cidx_body,
    grid=(M // MB,),
    in_specs=[pl.BlockSpec((NF, MB, 128), lambda i: (0, i, 0))],
    out_specs=pl.BlockSpec((MB, 128), lambda i: (i, 0)),
    out_shape=jax.ShapeDtypeStruct((M, 128), jnp.int32),
)


def _sc_body(table_hbm, cidx_hbm, out_hbm, idx_v, rows_v, sem):
    wid = lax.axis_index("s") * NC + lax.axis_index("c")
    base = wid * ROWS_W
    pltpu.sync_copy(cidx_hbm.at[pl.ds(base, ROWS_W)], idx_v)

    def step(g, carry):
        off = g * CHUNK
        pltpu.async_copy(table_hbm.at[idx_v.at[pl.ds(off, CHUNK)]], rows_v, sem).wait()
        pltpu.sync_copy(rows_v, out_hbm.at[pl.ds(base + off, CHUNK)])
        return carry

    lax.fori_loop(0, STEPS, step, 0)


import functools


@functools.cache
def _sc_call():
    return pl.kernel(
        _sc_body,
        out_type=jax.ShapeDtypeStruct((N_PIX, EMB), jnp.float32),
        mesh=plsc.VectorSubcoreMesh(core_axis_name="c", subcore_axis_name="s"),
        scratch_types=[
            pltpu.VMEM((ROWS_W,), jnp.int32),
            pltpu.VMEM((CHUNK, EMB), jnp.float32),
            pltpu.SemaphoreType.DMA,
        ],
    )


def kernel(edge_feature, tables):
    ef = edge_feature.astype(jnp.int32).reshape(N_PIX, NF)
    ef_t = ef.T.reshape(NF, M, 128)
    big = _table_call(tables.astype(jnp.float32))
    cidx = _cidx_call(ef_t).reshape(N_PIX)
    out = _sc_call()(big, cidx)
    return out.reshape(8, 256, 256, EMB)


# speedup vs baseline: 30.4617x; 1.3825x over previous
import jax
import jax.numpy as jnp
from jax import lax
from jax.experimental import pallas as pl
from jax.experimental.pallas import tpu as pltpu
from jax.experimental.pallas import tpu_sc as plsc

EMB = 128
NF = 4
NE = 10
NCOMB = NE ** NF
N_PIX = 8 * 256 * 256
NC = 2
NS = 16
NW = NC * NS
ROWS_W = N_PIX // NW
CHUNK = 128
STEPS = ROWS_W // CHUNK
M = N_PIX // 128
MB = 512


def _table_body(tab_ref, out_ref):
    rid = lax.broadcasted_iota(jnp.int32, (NCOMB, 1), 0)
    acc = jnp.zeros((NCOMB, EMB), jnp.float32)
    div = NCOMB // NE
    for f in range(NF):
        sub = (rid // div) % NE
        oh = (sub == lax.broadcasted_iota(jnp.int32, (NCOMB, NE), 1)).astype(jnp.float32)
        t = tab_ref[f]
        hi = t.astype(jnp.bfloat16).astype(jnp.float32)
        acc = acc + jnp.dot(oh, hi, preferred_element_type=jnp.float32)
        acc = acc + jnp.dot(oh, t - hi, preferred_element_type=jnp.float32)
        div //= NE
    out_ref[...] = acc


_table_call = pl.pallas_call(
    _table_body,
    out_shape=jax.ShapeDtypeStruct((NCOMB, EMB), jnp.float32),
)


def _cidx_body(ef_ref, out_ref):
    x = ef_ref[...]
    out_ref[...] = ((x[0] * NE + x[1]) * NE + x[2]) * NE + x[3]


_cidx_call = pl.pallas_call(
    _cidx_body,
    grid=(M // MB,),
    in_specs=[pl.BlockSpec((NF, MB, 128), lambda i: (0, i, 0))],
    out_specs=pl.BlockSpec((MB, 128), lambda i: (i, 0)),
    out_shape=jax.ShapeDtypeStruct((M, 128), jnp.int32),
)


NBUF = 4


def _sc_body(table_hbm, cidx_hbm, out_hbm, idx_v, rows_v, *sems):
    gsems, wsems = sems[:NBUF], sems[NBUF:]
    wid = lax.axis_index("s") * NC + lax.axis_index("c")
    base = wid * ROWS_W
    pltpu.sync_copy(cidx_hbm.at[pl.ds(base, ROWS_W)], idx_v)

    def gather(g, b):
        return pltpu.make_async_copy(
            table_hbm.at[idx_v.at[pl.ds(g * CHUNK, CHUNK)]], rows_v.at[b], gsems[b])

    def write(g, b):
        return pltpu.make_async_copy(
            rows_v.at[b], out_hbm.at[pl.ds(base + g * CHUNK, CHUNK)], wsems[b])

    for b in range(NBUF):
        gather(b, b).start()

    def body(go, carry):
        for b in range(NBUF):
            g = go * NBUF + b
            gather(g, b).wait()
            write(g, b).start()
        for b in range(NBUF):
            g = go * NBUF + b
            write(g, b).wait()

            @pl.when(g + NBUF < STEPS)
            def _():
                gather(g + NBUF, b).start()

        return carry

    lax.fori_loop(0, STEPS // NBUF, body, 0)


import functools


@functools.cache
def _sc_call():
    return pl.kernel(
        _sc_body,
        out_type=jax.ShapeDtypeStruct((N_PIX, EMB), jnp.float32),
        mesh=plsc.VectorSubcoreMesh(core_axis_name="c", subcore_axis_name="s"),
        scratch_types=[
            pltpu.VMEM((ROWS_W,), jnp.int32),
            pltpu.VMEM((NBUF, CHUNK, EMB), jnp.float32),
        ] + [pltpu.SemaphoreType.DMA] * (2 * NBUF),
    )


def kernel(edge_feature, tables):
    ef = edge_feature.astype(jnp.int32).reshape(N_PIX, NF)
    ef_t = ef.T.reshape(NF, M, 128)
    big = _table_call(tables.astype(jnp.float32))
    cidx = _cidx_call(ef_t).reshape(N_PIX)
    out = _sc_call()(big, cidx)
    return out.reshape(8, 256, 256, EMB)


# speedup vs baseline: 31.3037x; 1.0276x over previous
import jax
import jax.numpy as jnp
from jax import lax
from jax.experimental import pallas as pl
from jax.experimental.pallas import tpu as pltpu
from jax.experimental.pallas import tpu_sc as plsc

EMB = 128
NF = 4
NE = 10
NCOMB = NE ** NF
N_PIX = 8 * 256 * 256
NC = 2
NS = 16
NW = NC * NS
ROWS_W = N_PIX // NW
CHUNK = 128
STEPS = ROWS_W // CHUNK
M = N_PIX // 128
MB = 512


def _table_body(tab_ref, out_ref):
    rid = lax.broadcasted_iota(jnp.int32, (NCOMB, 1), 0)
    acc = jnp.zeros((NCOMB, EMB), jnp.float32)
    div = NCOMB // NE
    for f in range(NF):
        sub = (rid // div) % NE
        oh = (sub == lax.broadcasted_iota(jnp.int32, (NCOMB, NE), 1)).astype(jnp.float32)
        t = tab_ref[f]
        hi = t.astype(jnp.bfloat16).astype(jnp.float32)
        acc = acc + jnp.dot(oh, hi, preferred_element_type=jnp.float32)
        acc = acc + jnp.dot(oh, t - hi, preferred_element_type=jnp.float32)
        div //= NE
    out_ref[...] = acc


_table_call = pl.pallas_call(
    _table_body,
    out_shape=jax.ShapeDtypeStruct((NCOMB, EMB), jnp.float32),
)


def _cidx_body(ef_ref, out_ref):
    x = ef_ref[...]
    out_ref[...] = ((x[0] * NE + x[1]) * NE + x[2]) * NE + x[3]


_cidx_call = pl.pallas_call(
    _cidx_body,
    grid=(M // MB,),
    in_specs=[pl.BlockSpec((NF, MB, 128), lambda i: (0, i, 0))],
    out_specs=pl.BlockSpec((MB, 128), lambda i: (i, 0)),
    out_shape=jax.ShapeDtypeStruct((M, 128), jnp.int32),
)


NBUF = 4


def _sc_body(table_hbm, cidx_hbm, out_hbm, idx_v, rows_v, *sems):
    gsems, wsems = sems[:NBUF], sems[NBUF:]
    wid = lax.axis_index("s") * NC + lax.axis_index("c")
    base = wid * ROWS_W
    pltpu.sync_copy(cidx_hbm.at[pl.ds(base, ROWS_W)], idx_v)

    def gather(g, b):
        return pltpu.make_async_copy(
            table_hbm.at[idx_v.at[pl.ds(g * CHUNK, CHUNK)]], rows_v.at[b], gsems[b])

    def write(g, b):
        return pltpu.make_async_copy(
            rows_v.at[b], out_hbm.at[pl.ds(base + g * CHUNK, CHUNK)], wsems[b])

    LEAD = 2
    for g in range(LEAD):
        gather(g, g % NBUF).start()

    def body(ko, carry):
        for b in range(NBUF):
            k = ko * NBUF + b

            @pl.when(k >= LEAD)
            def _():
                write(k - LEAD, (b - LEAD) % NBUF).wait()

            @pl.when(k + LEAD < STEPS)
            def _():
                gather(k + LEAD, (b + LEAD) % NBUF).start()

            gather(k, b).wait()
            write(k, b).start()
        return carry

    lax.fori_loop(0, STEPS // NBUF, body, 0)
    for k in range(STEPS - LEAD, STEPS):
        write(k, k % NBUF).wait()


import functools


@functools.cache
def _sc_call():
    return pl.kernel(
        _sc_body,
        out_type=jax.ShapeDtypeStruct((N_PIX, EMB), jnp.float32),
        mesh=plsc.VectorSubcoreMesh(core_axis_name="c", subcore_axis_name="s"),
        scratch_types=[
            pltpu.VMEM((ROWS_W,), jnp.int32),
            pltpu.VMEM((NBUF, CHUNK, EMB), jnp.float32),
        ] + [pltpu.SemaphoreType.DMA] * (2 * NBUF),
    )


def kernel(edge_feature, tables):
    ef = edge_feature.astype(jnp.int32).reshape(N_PIX, NF)
    ef_t = ef.T.reshape(NF, M, 128)
    big = _table_call(tables.astype(jnp.float32))
    cidx = _cidx_call(ef_t).reshape(N_PIX)
    out = _sc_call()(big, cidx)
    return out.reshape(8, 256, 256, EMB)
